# baseline (speedup 1.0000x reference)
import functools

import jax
import jax.numpy as jnp
from jax import lax
from jax.experimental import pallas as pl
from jax.experimental.pallas import tpu as pltpu
from jax.experimental.pallas import tpu_sc as plsc

_B, _C, _H, _W = 4, 19, 512, 512
_HTC = 352
_HSC = _H - _HTC
_NW = 32
_WPB = _NW // _B
_RPW = _HSC // _WPB
_RPC = 2
_NCH = _RPW // _RPC
_G = (_RPC * _W) // 16
_HB = 32

_LN2 = 0.6931471805599453
_EPS = 1e-3


def _ln(s):
    bits = lax.bitcast_convert_type(s, jnp.int32)
    y = bits.astype(jnp.float32) * (_LN2 / 8388608.0) - ((127.0 - 0.0450466) * _LN2)
    for _ in range(2):
        y = y + s * jnp.exp(-y) - 1.0
    return y


def _sc_body(score_hbm, target_hbm, out_hbm, sbuf, tbuf, obuf,
             ss0, ss1, st0, st1):
    cid = lax.axis_index("c")
    sid = lax.axis_index("s")
    wid = sid * 2 + cid
    b = wid // _WPB
    wrow = _HTC + (wid % _WPB) * _RPW
    sems = ((ss0, st0), (ss1, st1))

    def start(j, par):
        r0 = wrow + j * _RPC
        pltpu.async_copy(score_hbm.at[b, :, pl.ds(r0, _RPC), :],
                         sbuf.at[par], sems[par][0])
        pltpu.async_copy(target_hbm.at[b, pl.ds(r0, _RPC), :],
                         tbuf.at[par], sems[par][1])

    def wait(par):
        pltpu.make_async_copy(score_hbm.at[b, :, pl.ds(wrow, _RPC), :],
                              sbuf.at[par], sems[par][0]).wait()
        pltpu.make_async_copy(target_hbm.at[b, pl.ds(wrow, _RPC), :],
                              tbuf.at[par], sems[par][1]).wait()

    def make_group(par):
        sref = sbuf.at[par]

        def group(i, accs):
            nll, cnt, saa, sbb, scc = accs
            r = lax.shift_right_logical(i, 5)
            col = lax.shift_left(jnp.bitwise_and(i, 31), 4)
            t = tbuf[par, r, pl.ds(col, 16)]

            def ld(c):
                return sbuf[par, c, r, pl.ds(col, 16)]

            hc = _C // 2
            m1 = ld(0)
            am1 = jnp.zeros((16,), jnp.float32)
            m2 = ld(hc)
            am2 = jnp.full((16,), float(hc), jnp.float32)
            for c in range(1, hc):
                x1 = ld(c)
                gt1 = x1 > m1
                m1 = jnp.where(gt1, x1, m1)
                am1 = jnp.where(gt1, jnp.float32(c), am1)
                x2 = ld(hc + c)
                gt2 = x2 > m2
                m2 = jnp.where(gt2, x2, m2)
                am2 = jnp.where(gt2, jnp.float32(hc + c), am2)
            xl = ld(_C - 1)
            gtl = xl > m2
            m2 = jnp.where(gtl, xl, m2)
            am2 = jnp.where(gtl, jnp.float32(_C - 1), am2)
            gt = m2 > m1
            m = jnp.where(gt, m2, m1)
            am = jnp.where(gt, am2, am1)
            s1 = jnp.exp(ld(0) - m)
            s2 = jnp.exp(ld(1) - m)
            for c in range(2, _C, 2):
                s1 = s1 + jnp.exp(ld(c) - m)
                if c + 1 < _C:
                    s2 = s2 + jnp.exp(ld(c + 1) - m)
            s = s1 + s2
            rvec = jnp.full((16,), r, jnp.int32)
            cvec = col + lax.iota(jnp.int32, 16)
            t0 = jnp.maximum(t, 0)
            xt = plsc.load_gather(sref, [t0, rvec, cvec])
            lse = _ln(s) + m
            valid = t >= 0
            vf = jnp.where(valid, 1.0, 0.0).astype(jnp.float32)
            tf = t.astype(jnp.float32)
            nll = nll + jnp.where(valid, lse - xt, 0.0)
            cnt = cnt + vf
            saa = saa + am * tf
            sbb = sbb + am * am
            scc = scc + tf * tf
            return (nll, cnt, saa, sbb, scc)

        return group

    start(0, 0)
    start(1, 1)

    def pair(j2, accs):
        j = j2 * 2
        for par in range(2):
            jj = j + par
            wait(par)
            accs = lax.fori_loop(0, _G, make_group(par), accs, unroll=False)
            start(jnp.minimum(jj + 2, _NCH - 1), par)
        return accs

    zeros = jnp.zeros((16,), jnp.float32)
    accs = lax.fori_loop(0, _NCH // 2, pair,
                         (zeros, zeros, zeros, zeros, zeros), unroll=False)
    if _NCH % 2 == 1:
        wait(0)
        accs = lax.fori_loop(0, _G, make_group(0), accs, unroll=False)
        wait(1)
    else:
        wait(0)
        wait(1)
    for q in range(5):
        obuf[q, :] = accs[q]
    pltpu.sync_copy(obuf, out_hbm.at[wid])


def _tc_body(score_ref, target_ref, out_ref):
    a_nll = jnp.zeros((8, _W), jnp.float32)
    a_vf = jnp.zeros((8, _W), jnp.float32)
    a_aa = jnp.zeros((8, _W), jnp.float32)
    a_bb = jnp.zeros((8, _W), jnp.float32)
    a_cc = jnp.zeros((8, _W), jnp.float32)
    for r8 in range(_HB // 8):
        sl = pl.ds(r8 * 8, 8)
        t = target_ref[0, sl, :]
        valid = t >= 0
        t0 = jnp.where(valid, t, 0)
        x0 = score_ref[0, 0, sl, :]
        m = x0
        am = jnp.zeros((8, _W), jnp.float32)
        xt = jnp.where(t0 == 0, x0, 0.0)
        for c in range(1, _C):
            xc = score_ref[0, c, sl, :]
            gt = xc > m
            m = jnp.where(gt, xc, m)
            am = jnp.where(gt, jnp.float32(c), am)
            xt = jnp.where(t0 == c, xc, xt)
        s = jnp.exp(x0 - m)
        for c in range(1, _C):
            s = s + jnp.exp(score_ref[0, c, sl, :] - m)
        lse = jnp.log(s) + m
        vf = valid.astype(jnp.float32)
        tf = t.astype(jnp.float32)
        a_nll = a_nll + jnp.where(valid, lse - xt, 0.0)
        a_vf = a_vf + vf
        a_aa = a_aa + am * tf
        a_bb = a_bb + am * am
        a_cc = a_cc + tf * tf
    out_ref[0, 0, 0, 0] = jnp.sum(a_nll)
    out_ref[0, 0, 0, 1] = jnp.sum(a_vf)
    out_ref[0, 0, 0, 2] = jnp.sum(a_aa)
    out_ref[0, 0, 0, 3] = jnp.sum(a_bb)
    out_ref[0, 0, 0, 4] = jnp.sum(a_cc)


@jax.jit
def _run(score, target):
    mesh = plsc.VectorSubcoreMesh(core_axis_name="c", subcore_axis_name="s")
    call = pl.kernel(
        _sc_body,
        out_type=jax.ShapeDtypeStruct((_NW, 5, 16), jnp.float32),
        mesh=mesh,
        scratch_types=[
            pltpu.VMEM((2, _C, _RPC, _W), jnp.float32),
            pltpu.VMEM((2, _RPC, _W), jnp.int32),
            pltpu.VMEM((5, 16), jnp.float32),
            pltpu.SemaphoreType.DMA,
            pltpu.SemaphoreType.DMA,
            pltpu.SemaphoreType.DMA,
            pltpu.SemaphoreType.DMA,
        ],
        compiler_params=pltpu.CompilerParams(needs_layout_passes=False),
    )
    sc_part = call(score, target)

    nblk = _HTC // _HB
    tc_part = pl.pallas_call(
        _tc_body,
        grid=(_B, nblk),
        in_specs=[
            pl.BlockSpec((1, _C, _HB, _W), lambda b, j: (b, 0, j, 0)),
            pl.BlockSpec((1, _HB, _W), lambda b, j: (b, j, 0)),
        ],
        out_specs=pl.BlockSpec((1, 1, 1, 5), lambda b, j: (b, j, 0, 0),
                               memory_space=pltpu.SMEM),
        out_shape=jax.ShapeDtypeStruct((_B, nblk, 1, 5), jnp.float32),
    )(score, target)

    part = sc_part.sum(axis=2)
    per_b = (part.reshape(_B, _WPB, 5).sum(axis=1)
             + tc_part.sum(axis=(1, 2)))
    nll_tot = per_b[:, 0].sum()
    cnt_tot = per_b[:, 1].sum()
    ce = nll_tot / jnp.maximum(cnt_tot, 1.0)
    a = per_b[:, 2]
    bb = per_b[:, 3] + _EPS
    cc = per_b[:, 4] + _EPS
    dice = 1.0 - 2.0 * a / (bb + cc)
    return ce + dice


def kernel(score, target, epoch):
    return _run(score, target)


# speedup vs baseline: 1.1484x; 1.1484x over previous
import functools

import jax
import jax.numpy as jnp
from jax import lax
from jax.experimental import pallas as pl
from jax.experimental.pallas import tpu as pltpu
from jax.experimental.pallas import tpu_sc as plsc

_B, _C, _H, _W = 4, 19, 512, 512
_HTC = 320
_HSC = _H - _HTC
_NW = 32
_WPB = _NW // _B
_RPW = _HSC // _WPB
_RPC = 2
_NCH = _RPW // _RPC
_G = (_RPC * _W) // 16
_HB = 64

_LN2 = 0.6931471805599453
_EPS = 1e-3


def _ln(s):
    bits = lax.bitcast_convert_type(s, jnp.int32)
    y = bits.astype(jnp.float32) * (_LN2 / 8388608.0) - ((127.0 - 0.0450466) * _LN2)
    for _ in range(2):
        y = y + s * jnp.exp(-y) - 1.0
    return y


def _sc_body(score_hbm, target_hbm, out_hbm, sbuf, tbuf, obuf, ss0, ss1):
    cid = lax.axis_index("c")
    sid = lax.axis_index("s")
    wid = sid * 2 + cid
    b = wid // _WPB
    wrow = _HTC + (wid % _WPB) * _RPW
    sems = (ss0, ss1)

    def start(j, par):
        r0 = wrow + j * _RPC
        pltpu.async_copy(score_hbm.at[b, :, pl.ds(r0, _RPC), :],
                         sbuf.at[par], sems[par])

    def wait(par):
        pltpu.make_async_copy(score_hbm.at[b, :, pl.ds(wrow, _RPC), :],
                              sbuf.at[par], sems[par]).wait()

    def make_group(par, j):
        sref = sbuf.at[par]

        def group(i, accs):
            nll, cnt, saa, sbb, scc = accs
            r = lax.shift_right_logical(i, 5)
            col = lax.shift_left(jnp.bitwise_and(i, 31), 4)
            t = tbuf[j * _RPC + r, pl.ds(col, 16)]

            def ld(c):
                return sbuf[par, c, r, pl.ds(col, 16)]

            hc = _C // 2
            m1 = ld(0)
            am1 = jnp.zeros((16,), jnp.float32)
            m2 = ld(hc)
            am2 = jnp.full((16,), float(hc), jnp.float32)
            for c in range(1, hc):
                x1 = ld(c)
                gt1 = x1 > m1
                m1 = jnp.where(gt1, x1, m1)
                am1 = jnp.where(gt1, jnp.float32(c), am1)
                x2 = ld(hc + c)
                gt2 = x2 > m2
                m2 = jnp.where(gt2, x2, m2)
                am2 = jnp.where(gt2, jnp.float32(hc + c), am2)
            xl = ld(_C - 1)
            gtl = xl > m2
            m2 = jnp.where(gtl, xl, m2)
            am2 = jnp.where(gtl, jnp.float32(_C - 1), am2)
            gt = m2 > m1
            m = jnp.where(gt, m2, m1)
            am = jnp.where(gt, am2, am1)
            s1 = jnp.exp(ld(0) - m)
            s2 = jnp.exp(ld(1) - m)
            for c in range(2, _C, 2):
                s1 = s1 + jnp.exp(ld(c) - m)
                if c + 1 < _C:
                    s2 = s2 + jnp.exp(ld(c + 1) - m)
            s = s1 + s2
            rvec = jnp.full((16,), r, jnp.int32)
            cvec = col + lax.iota(jnp.int32, 16)
            t0 = jnp.maximum(t, 0)
            xt = plsc.load_gather(sref, [t0, rvec, cvec])
            lse = _ln(s) + m
            valid = t >= 0
            vf = jnp.where(valid, 1.0, 0.0).astype(jnp.float32)
            tf = t.astype(jnp.float32)
            nll = nll + jnp.where(valid, lse - xt, 0.0)
            cnt = cnt + vf
            saa = saa + am * tf
            sbb = sbb + am * am
            scc = scc + tf * tf
            return (nll, cnt, saa, sbb, scc)

        return group

    start(0, 0)
    start(1, 1)
    pltpu.sync_copy(target_hbm.at[b, pl.ds(wrow, _RPW), :], tbuf)

    def pair(j2, accs):
        j = j2 * 2
        for par in range(2):
            jj = j + par
            wait(par)
            accs = lax.fori_loop(0, _G, make_group(par, jj), accs,
                                 unroll=False)
            start(jnp.minimum(jj + 2, _NCH - 1), par)
        return accs

    zeros = jnp.zeros((16,), jnp.float32)
    accs = lax.fori_loop(0, _NCH // 2, pair,
                         (zeros, zeros, zeros, zeros, zeros), unroll=False)
    if _NCH % 2 == 1:
        wait(0)
        accs = lax.fori_loop(0, _G, make_group(0, _NCH - 1), accs,
                             unroll=False)
        wait(1)
    else:
        wait(0)
        wait(1)
    for q in range(5):
        obuf[q, :] = accs[q]
    pltpu.sync_copy(obuf, out_hbm.at[wid])


def _tc_body(score_ref, target_ref, out_ref):
    a_nll = jnp.zeros((8, _W), jnp.float32)
    a_vf = jnp.zeros((8, _W), jnp.float32)
    a_aa = jnp.zeros((8, _W), jnp.float32)
    a_bb = jnp.zeros((8, _W), jnp.float32)
    a_cc = jnp.zeros((8, _W), jnp.float32)
    for r8 in range(_HB // 8):
        sl = pl.ds(r8 * 8, 8)
        t = target_ref[0, sl, :]
        valid = t >= 0
        t0 = jnp.where(valid, t, 0)
        x0 = score_ref[0, 0, sl, :]
        m = x0
        am = jnp.zeros((8, _W), jnp.float32)
        xt = jnp.where(t0 == 0, x0, 0.0)
        for c in range(1, _C):
            xc = score_ref[0, c, sl, :]
            gt = xc > m
            m = jnp.where(gt, xc, m)
            am = jnp.where(gt, jnp.float32(c), am)
            xt = jnp.where(t0 == c, xc, xt)
        s = jnp.exp(x0 - m)
        for c in range(1, _C):
            s = s + jnp.exp(score_ref[0, c, sl, :] - m)
        lse = jnp.log(s) + m
        vf = valid.astype(jnp.float32)
        tf = t.astype(jnp.float32)
        a_nll = a_nll + jnp.where(valid, lse - xt, 0.0)
        a_vf = a_vf + vf
        a_aa = a_aa + am * tf
        a_bb = a_bb + am * am
        a_cc = a_cc + tf * tf
    out_ref[0, 0, 0, 0] = jnp.sum(a_nll)
    out_ref[0, 0, 0, 1] = jnp.sum(a_vf)
    out_ref[0, 0, 0, 2] = jnp.sum(a_aa)
    out_ref[0, 0, 0, 3] = jnp.sum(a_bb)
    out_ref[0, 0, 0, 4] = jnp.sum(a_cc)


@jax.jit
def _run(score, target):
    mesh = plsc.VectorSubcoreMesh(core_axis_name="c", subcore_axis_name="s")
    call = pl.kernel(
        _sc_body,
        out_type=jax.ShapeDtypeStruct((_NW, 5, 16), jnp.float32),
        mesh=mesh,
        scratch_types=[
            pltpu.VMEM((2, _C, _RPC, _W), jnp.float32),
            pltpu.VMEM((_RPW, _W), jnp.int32),
            pltpu.VMEM((5, 16), jnp.float32),
            pltpu.SemaphoreType.DMA,
            pltpu.SemaphoreType.DMA,
        ],
        compiler_params=pltpu.CompilerParams(needs_layout_passes=False),
    )
    sc_part = call(score, target)

    nblk = _HTC // _HB
    tc_part = pl.pallas_call(
        _tc_body,
        grid=(_B, nblk),
        in_specs=[
            pl.BlockSpec((1, _C, _HB, _W), lambda b, j: (b, 0, j, 0)),
            pl.BlockSpec((1, _HB, _W), lambda b, j: (b, j, 0)),
        ],
        out_specs=pl.BlockSpec((1, 1, 1, 5), lambda b, j: (b, j, 0, 0),
                               memory_space=pltpu.SMEM),
        out_shape=jax.ShapeDtypeStruct((_B, nblk, 1, 5), jnp.float32),
    )(score, target)

    part = sc_part.sum(axis=2)
    per_b = (part.reshape(_B, _WPB, 5).sum(axis=1)
             + tc_part.sum(axis=(1, 2)))
    nll_tot = per_b[:, 0].sum()
    cnt_tot = per_b[:, 1].sum()
    ce = nll_tot / jnp.maximum(cnt_tot, 1.0)
    a = per_b[:, 2]
    bb = per_b[:, 3] + _EPS
    cc = per_b[:, 4] + _EPS
    dice = 1.0 - 2.0 * a / (bb + cc)
    return ce + dice


def kernel(score, target, epoch):
    return _run(score, target)


# speedup vs baseline: 1.1745x; 1.0227x over previous
import functools

import jax
import jax.numpy as jnp
from jax import lax
from jax.experimental import pallas as pl
from jax.experimental.pallas import tpu as pltpu
from jax.experimental.pallas import tpu_sc as plsc

_B, _C, _H, _W = 4, 19, 512, 512
_HTC = 320
_HSC = _H - _HTC
_NW = 32
_WPB = _NW // _B
_RPW = _HSC // _WPB
_RPC = 2
_NCH = _RPW // _RPC
_G = (_RPC * _W) // 16
_HB = 64

_LN2 = 0.6931471805599453
_EPS = 1e-3


def _ln(s):
    bits = lax.bitcast_convert_type(s, jnp.int32)
    y = bits.astype(jnp.float32) * (_LN2 / 8388608.0) - ((127.0 - 0.0450466) * _LN2)
    for _ in range(2):
        y = y + s * jnp.exp(-y) - 1.0
    return y


def _sc_body(score_hbm, target_hbm, out_hbm, sbuf, tbuf, obuf,
             ss0, ss1, st0, st1):
    cid = lax.axis_index("c")
    sid = lax.axis_index("s")
    wid = sid * 2 + cid
    b = wid // _WPB
    wrow = _HTC + (wid % _WPB) * _RPW
    sems = ((ss0, st0), (ss1, st1))

    def start(j, par):
        r0 = wrow + j * _RPC
        pltpu.async_copy(score_hbm.at[b, :, pl.ds(r0, _RPC), :],
                         sbuf.at[par], sems[par][0])
        pltpu.async_copy(target_hbm.at[b, pl.ds(r0, _RPC), :],
                         tbuf.at[par], sems[par][1])

    def wait(par):
        pltpu.make_async_copy(score_hbm.at[b, :, pl.ds(wrow, _RPC), :],
                              sbuf.at[par], sems[par][0]).wait()
        pltpu.make_async_copy(target_hbm.at[b, pl.ds(wrow, _RPC), :],
                              tbuf.at[par], sems[par][1]).wait()

    def make_group(par):
        sref = sbuf.at[par]

        def group(i, accs):
            nll, cnt, saa, sbb, scc = accs
            r = lax.shift_right_logical(i, 5)
            col = lax.shift_left(jnp.bitwise_and(i, 31), 4)
            t = tbuf[par, r, pl.ds(col, 16)]

            def ld(c):
                return sbuf[par, c, r, pl.ds(col, 16)]

            hc = _C // 2
            m1 = ld(0)
            am1 = jnp.zeros((16,), jnp.float32)
            m2 = ld(hc)
            am2 = jnp.full((16,), float(hc), jnp.float32)
            for c in range(1, hc):
                x1 = ld(c)
                gt1 = x1 > m1
                m1 = jnp.where(gt1, x1, m1)
                am1 = jnp.where(gt1, jnp.float32(c), am1)
                x2 = ld(hc + c)
                gt2 = x2 > m2
                m2 = jnp.where(gt2, x2, m2)
                am2 = jnp.where(gt2, jnp.float32(hc + c), am2)
            xl = ld(_C - 1)
            gtl = xl > m2
            m2 = jnp.where(gtl, xl, m2)
            am2 = jnp.where(gtl, jnp.float32(_C - 1), am2)
            gt = m2 > m1
            m = jnp.where(gt, m2, m1)
            am = jnp.where(gt, am2, am1)
            s1 = jnp.exp(ld(0) - m)
            s2 = jnp.exp(ld(1) - m)
            for c in range(2, _C, 2):
                s1 = s1 + jnp.exp(ld(c) - m)
                if c + 1 < _C:
                    s2 = s2 + jnp.exp(ld(c + 1) - m)
            s = s1 + s2
            rvec = jnp.full((16,), r, jnp.int32)
            cvec = col + lax.iota(jnp.int32, 16)
            t0 = jnp.maximum(t, 0)
            xt = plsc.load_gather(sref, [t0, rvec, cvec])
            lse = _ln(s) + m
            valid = t >= 0
            vf = jnp.where(valid, 1.0, 0.0).astype(jnp.float32)
            tf = t.astype(jnp.float32)
            nll = nll + jnp.where(valid, lse - xt, 0.0)
            cnt = cnt + vf
            saa = saa + am * tf
            sbb = sbb + am * am
            scc = scc + tf * tf
            return (nll, cnt, saa, sbb, scc)

        return group

    start(0, 0)
    start(1, 1)

    def pair(j2, accs):
        j = j2 * 2
        for par in range(2):
            jj = j + par
            wait(par)
            accs = lax.fori_loop(0, _G, make_group(par), accs, unroll=False)
            start(jnp.minimum(jj + 2, _NCH - 1), par)
        return accs

    zeros = jnp.zeros((16,), jnp.float32)
    accs = lax.fori_loop(0, _NCH // 2, pair,
                         (zeros, zeros, zeros, zeros, zeros), unroll=False)
    if _NCH % 2 == 1:
        wait(0)
        accs = lax.fori_loop(0, _G, make_group(0), accs, unroll=False)
        wait(1)
    else:
        wait(0)
        wait(1)
    for q in range(5):
        obuf[q, :] = accs[q]
    pltpu.sync_copy(obuf, out_hbm.at[wid])


def _tc_body(score_ref, target_ref, out_ref):
    a_nll = jnp.zeros((8, _W), jnp.float32)
    a_vf = jnp.zeros((8, _W), jnp.float32)
    a_aa = jnp.zeros((8, _W), jnp.float32)
    a_bb = jnp.zeros((8, _W), jnp.float32)
    a_cc = jnp.zeros((8, _W), jnp.float32)
    for r8 in range(_HB // 8):
        sl = pl.ds(r8 * 8, 8)
        t = target_ref[0, sl, :]
        valid = t >= 0
        t0 = jnp.where(valid, t, 0)
        x0 = score_ref[0, 0, sl, :]
        m = x0
        am = jnp.zeros((8, _W), jnp.float32)
        xt = jnp.where(t0 == 0, x0, 0.0)
        for c in range(1, _C):
            xc = score_ref[0, c, sl, :]
            gt = xc > m
            m = jnp.where(gt, xc, m)
            am = jnp.where(gt, jnp.float32(c), am)
            xt = jnp.where(t0 == c, xc, xt)
        s = jnp.exp(x0 - m)
        for c in range(1, _C):
            s = s + jnp.exp(score_ref[0, c, sl, :] - m)
        lse = jnp.log(s) + m
        vf = valid.astype(jnp.float32)
        tf = t.astype(jnp.float32)
        a_nll = a_nll + jnp.where(valid, lse - xt, 0.0)
        a_vf = a_vf + vf
        a_aa = a_aa + am * tf
        a_bb = a_bb + am * am
        a_cc = a_cc + tf * tf
    out_ref[0, 0, 0, 0] = jnp.sum(a_nll)
    out_ref[0, 0, 0, 1] = jnp.sum(a_vf)
    out_ref[0, 0, 0, 2] = jnp.sum(a_aa)
    out_ref[0, 0, 0, 3] = jnp.sum(a_bb)
    out_ref[0, 0, 0, 4] = jnp.sum(a_cc)


@jax.jit
def _run(score, target):
    mesh = plsc.VectorSubcoreMesh(core_axis_name="c", subcore_axis_name="s")
    call = pl.kernel(
        _sc_body,
        out_type=jax.ShapeDtypeStruct((_NW, 5, 16), jnp.float32),
        mesh=mesh,
        scratch_types=[
            pltpu.VMEM((2, _C, _RPC, _W), jnp.float32),
            pltpu.VMEM((2, _RPC, _W), jnp.int32),
            pltpu.VMEM((5, 16), jnp.float32),
            pltpu.SemaphoreType.DMA,
            pltpu.SemaphoreType.DMA,
            pltpu.SemaphoreType.DMA,
            pltpu.SemaphoreType.DMA,
        ],
        compiler_params=pltpu.CompilerParams(needs_layout_passes=False),
    )
    sc_part = call(score, target)

    nblk = _HTC // _HB
    tc_part = pl.pallas_call(
        _tc_body,
        grid=(_B, nblk),
        in_specs=[
            pl.BlockSpec((1, _C, _HB, _W), lambda b, j: (b, 0, j, 0)),
            pl.BlockSpec((1, _HB, _W), lambda b, j: (b, j, 0)),
        ],
        out_specs=pl.BlockSpec((1, 1, 1, 5), lambda b, j: (b, j, 0, 0),
                               memory_space=pltpu.SMEM),
        out_shape=jax.ShapeDtypeStruct((_B, nblk, 1, 5), jnp.float32),
    )(score, target)

    part = sc_part.sum(axis=2)
    per_b = (part.reshape(_B, _WPB, 5).sum(axis=1)
             + tc_part.sum(axis=(1, 2)))
    nll_tot = per_b[:, 0].sum()
    cnt_tot = per_b[:, 1].sum()
    ce = nll_tot / jnp.maximum(cnt_tot, 1.0)
    a = per_b[:, 2]
    bb = per_b[:, 3] + _EPS
    cc = per_b[:, 4] + _EPS
    dice = 1.0 - 2.0 * a / (bb + cc)
    return ce + dice


def kernel(score, target, epoch):
    return _run(score, target)
